# baseline (speedup 1.0000x reference)
import functools
import math

import jax
import jax.numpy as jnp
from jax import lax
from jax.experimental import pallas as pl
from jax.experimental.pallas import tpu as pltpu
from jax.experimental.pallas import tpu_sc as plsc

DIM = 64
ROW_W = 128

_NC = 2
_NS = 16
_NW = _NC * _NS

_CHUNK = 256
_IDX_W = 128


def _sc_gather(table128, idx, n_rows):
    per_w = n_rows // _NW
    n_chunks = per_w // _CHUNK
    n_streams = _CHUNK // _IDX_W

    mesh = plsc.VectorSubcoreMesh(
        core_axis_name="c", subcore_axis_name="s",
        num_cores=_NC, num_subcores=_NS)

    @functools.partial(
        pl.kernel,
        mesh=mesh,
        out_type=jax.ShapeDtypeStruct((n_rows, ROW_W), jnp.float32),
        scratch_types=[
            pltpu.VMEM((2, _CHUNK), jnp.int32),
            pltpu.VMEM((2, _CHUNK, ROW_W), jnp.float32),
            pltpu.SemaphoreType.DMA,
            pltpu.SemaphoreType.DMA,
        ],
        compiler_params=pltpu.CompilerParams(use_tc_tiling_on_sc=False),
    )
    def k(table_hbm, idx_hbm, out_hbm, idx_v, rows_v, sem_g, sem_o):
        wid = lax.axis_index("s") * _NC + lax.axis_index("c")
        wbase = wid * per_w

        def fire_gather(c):
            p = lax.rem(c, 2)
            base = wbase + c * _CHUNK
            pltpu.sync_copy(idx_hbm.at[pl.ds(base, _CHUNK)], idx_v.at[p])
            for j in range(n_streams):
                pltpu.async_copy(
                    table_hbm.at[idx_v.at[p, pl.ds(j * _IDX_W, _IDX_W)]],
                    rows_v.at[p, pl.ds(j * _IDX_W, _IDX_W)],
                    sem_g)

        def wait_gather(c):
            p = lax.rem(c, 2)
            for j in range(n_streams):
                pltpu.make_async_copy(
                    table_hbm.at[idx_v.at[p, pl.ds(j * _IDX_W, _IDX_W)]],
                    rows_v.at[p, pl.ds(j * _IDX_W, _IDX_W)],
                    sem_g).wait()

        def drain_out():
            pltpu.make_async_copy(
                rows_v.at[0], out_hbm.at[pl.ds(wbase, _CHUNK)],
                sem_o).wait()

        def chunk_body(c, _):
            @pl.when(c >= 1)
            def _():
                drain_out()

            @pl.when(c + 1 < n_chunks)
            def _():
                fire_gather(c + 1)

            wait_gather(c)

            p = lax.rem(c, 2)
            base = wbase + c * _CHUNK
            pltpu.async_copy(
                rows_v.at[p], out_hbm.at[pl.ds(base, _CHUNK)], sem_o)
            return ()

        fire_gather(0)
        lax.fori_loop(0, n_chunks, chunk_body, (), unroll=False)
        drain_out()

    return k(table128, idx)


def _ln_body(emb_ref, pe_ref, g_ref, b_ref, out_ref):
    e = emb_ref[..., :DIM] + pe_ref[...]
    mu = jnp.mean(e, axis=-1, keepdims=True)
    var = jnp.mean(jnp.square(e - mu), axis=-1, keepdims=True)
    y = (e - mu) * lax.rsqrt(var + 1e-5) * g_ref[...] + b_ref[...]
    out_ref[...] = jnp.transpose(y, (1, 2, 0))


def _tc_ln(emb, pe, gamma, beta):
    b, l, d2 = emb.shape
    d = DIM
    lb = 8
    return pl.pallas_call(
        _ln_body,
        grid=(l // lb,),
        in_specs=[
            pl.BlockSpec((b, lb, d2), lambda i: (0, i, 0)),
            pl.BlockSpec((1, lb, d), lambda i: (0, i, 0)),
            pl.BlockSpec((1, 1, d), lambda i: (0, 0, 0)),
            pl.BlockSpec((1, 1, d), lambda i: (0, 0, 0)),
        ],
        out_specs=pl.BlockSpec((lb, d, b), lambda i: (i, 0, 0)),
        out_shape=jax.ShapeDtypeStruct((l, d, b), jnp.float32),
    )(emb, pe, gamma, beta)


def _pe_table(length, d):
    position = jnp.arange(length, dtype=jnp.float32)[:, None]
    div_term = jnp.exp(
        jnp.arange(0, d, 2, dtype=jnp.float32) * (-math.log(10000.0) / d))
    ang = position * div_term
    return jnp.stack([jnp.sin(ang), jnp.cos(ang)], axis=-1).reshape(length, d)


def kernel(x, word_embeddings_weight, ln_gamma, ln_beta):
    b, l = x.shape
    n = b * l
    table128 = jnp.pad(word_embeddings_weight, ((0, 0), (0, ROW_W - DIM)))
    gathered = _sc_gather(table128, x.reshape(n), n)
    pe = _pe_table(l, DIM)[None]
    g = ln_gamma.reshape(1, 1, DIM)
    be = ln_beta.reshape(1, 1, DIM)
    out_t = _tc_ln(gathered.reshape(b, l, ROW_W), pe, g, be)
    return out_t.transpose(2, 0, 1)


# speedup vs baseline: 1.4380x; 1.4380x over previous
import functools
import math

import jax
import jax.numpy as jnp
from jax import lax
from jax.experimental import pallas as pl
from jax.experimental.pallas import tpu as pltpu
from jax.experimental.pallas import tpu_sc as plsc

DIM = 64
ROW_W = 128

_NC = 2
_NS = 16
_NW = _NC * _NS

_CHUNK = 256
_IDX_W = 128


def _sc_gather(table128, idx, n_rows):
    per_w = n_rows // _NW
    n_chunks = per_w // _CHUNK
    n_streams = _CHUNK // _IDX_W

    mesh = plsc.VectorSubcoreMesh(
        core_axis_name="c", subcore_axis_name="s",
        num_cores=_NC, num_subcores=_NS)

    @functools.partial(
        pl.kernel,
        mesh=mesh,
        out_type=jax.ShapeDtypeStruct((n_rows, ROW_W), jnp.float32),
        scratch_types=[
            pltpu.VMEM((2, _CHUNK), jnp.int32),
            pltpu.VMEM((2, _CHUNK, ROW_W), jnp.float32),
            pltpu.SemaphoreType.DMA,
            pltpu.SemaphoreType.DMA,
        ],
        compiler_params=pltpu.CompilerParams(use_tc_tiling_on_sc=False),
    )
    def k(table_hbm, idx_hbm, out_hbm, idx_v, rows_v, sem_g, sem_o):
        wid = lax.axis_index("s") * _NC + lax.axis_index("c")
        wbase = wid * per_w

        def fire_gather(c):
            p = lax.rem(c, 2)
            base = wbase + c * _CHUNK
            pltpu.sync_copy(idx_hbm.at[pl.ds(base, _CHUNK)], idx_v.at[p])
            for j in range(n_streams):
                pltpu.async_copy(
                    table_hbm.at[idx_v.at[p, pl.ds(j * _IDX_W, _IDX_W)]],
                    rows_v.at[p, pl.ds(j * _IDX_W, _IDX_W)],
                    sem_g)

        def wait_gather(c):
            p = lax.rem(c, 2)
            for j in range(n_streams):
                pltpu.make_async_copy(
                    table_hbm.at[idx_v.at[p, pl.ds(j * _IDX_W, _IDX_W)]],
                    rows_v.at[p, pl.ds(j * _IDX_W, _IDX_W)],
                    sem_g).wait()

        def drain_out():
            pltpu.make_async_copy(
                rows_v.at[0], out_hbm.at[pl.ds(wbase, _CHUNK)],
                sem_o).wait()

        def chunk_body(c, _):
            @pl.when(c >= 1)
            def _():
                drain_out()

            @pl.when(c + 1 < n_chunks)
            def _():
                fire_gather(c + 1)

            wait_gather(c)

            p = lax.rem(c, 2)
            base = wbase + c * _CHUNK
            pltpu.async_copy(
                rows_v.at[p], out_hbm.at[pl.ds(base, _CHUNK)], sem_o)
            return ()

        fire_gather(0)
        lax.fori_loop(0, n_chunks, chunk_body, (), unroll=False)
        drain_out()

    return k(table128, idx)


def _ln_body(emb_ref, pe_ref, g_ref, b_ref, out_ref):
    e = emb_ref[..., :DIM] + pe_ref[...]
    mu = jnp.mean(e, axis=-1, keepdims=True)
    var = jnp.mean(jnp.square(e - mu), axis=-1, keepdims=True)
    out_ref[...] = (e - mu) * lax.rsqrt(var + 1e-5) * g_ref[...] + b_ref[...]


def _tc_ln(emb, pe, gamma, beta):
    b, l, d2 = emb.shape
    d = DIM
    bb = 32
    return pl.pallas_call(
        _ln_body,
        grid=(b // bb,),
        in_specs=[
            pl.BlockSpec((bb, l, d2), lambda i: (i, 0, 0)),
            pl.BlockSpec((1, l, d), lambda i: (0, 0, 0)),
            pl.BlockSpec((1, 1, d), lambda i: (0, 0, 0)),
            pl.BlockSpec((1, 1, d), lambda i: (0, 0, 0)),
        ],
        out_specs=pl.BlockSpec((bb, l, d), lambda i: (i, 0, 0)),
        out_shape=jax.ShapeDtypeStruct((b, l, d), jnp.float32),
    )(emb, pe, gamma, beta)


def _pe_table(length, d):
    position = jnp.arange(length, dtype=jnp.float32)[:, None]
    div_term = jnp.exp(
        jnp.arange(0, d, 2, dtype=jnp.float32) * (-math.log(10000.0) / d))
    ang = position * div_term
    return jnp.stack([jnp.sin(ang), jnp.cos(ang)], axis=-1).reshape(length, d)


def kernel(x, word_embeddings_weight, ln_gamma, ln_beta):
    b, l = x.shape
    n = b * l
    table128 = jnp.pad(word_embeddings_weight, ((0, 0), (0, ROW_W - DIM)))
    gathered = _sc_gather(table128, x.reshape(n), n)
    pe = _pe_table(l, DIM)[None]
    g = ln_gamma.reshape(1, 1, DIM)
    be = ln_beta.reshape(1, 1, DIM)
    return _tc_ln(gathered.reshape(b, l, ROW_W), pe, g, be)
